# baseline (speedup 1.0000x reference)
import jax
import jax.numpy as jnp
from jax.experimental import pallas as pl
from jax.experimental.pallas import tpu as pltpu

TOP_K = 8


def _routing(cls_ref, key_ref, ids_smem, rs_ref):
    eps = 1e-12
    k = key_ref[...]
    kn = jnp.sqrt(jnp.sum(k * k, axis=1, keepdims=True))
    k_n = k / jnp.maximum(kn, eps)
    x = cls_ref[...]
    xn = jnp.sqrt(jnp.sum(x * x, axis=1, keepdims=True))
    x_n = x / jnp.maximum(xn, eps)
    sim0 = jax.lax.dot_general(
        x_n, k_n, (((1,), (1,)), ((), ())),
        preferred_element_type=jnp.float32)
    B, P = sim0.shape

    col = jax.lax.broadcasted_iota(jnp.int32, (B, P), 1)
    sim = sim0
    counts2d = jnp.zeros((B, P), jnp.int32)
    for _ in range(TOP_K):
        m = jnp.max(sim, axis=1, keepdims=True)
        cand = jnp.where(sim == m, col, P)
        j = jnp.min(cand, axis=1, keepdims=True)
        oh = col == j
        counts2d = counts2d + oh.astype(jnp.int32)
        sim = jnp.where(oh, -jnp.inf, sim)

    cnt = jnp.sum(counts2d, axis=0, keepdims=True)
    p_row = jax.lax.broadcasted_iota(jnp.int32, (1, P), 1)
    key2 = (cnt * (2 * P) + (P - 1 - p_row)).astype(jnp.float32)
    ri = jax.lax.broadcasted_iota(jnp.int32, (P, P), 0)
    ci = jax.lax.broadcasted_iota(jnp.int32, (P, P), 1)
    ident = (ri == ci).astype(jnp.float32)
    key2_col = jax.lax.dot_general(
        ident, key2, (((1,), (1,)), ((), ())),
        preferred_element_type=jnp.float32)
    gt = (key2_col > key2).astype(jnp.int32)
    rank = jnp.sum(gt, axis=0, keepdims=True)
    for j in range(TOP_K):
        ids_smem[0, j] = jnp.sum(jnp.where(rank == j, p_row, 0))
    colsum = jnp.sum(sim0, axis=0, keepdims=True)
    sel = (rank < TOP_K).astype(jnp.float32)
    rs_ref[0, 0] = jnp.sum(colsum * sel) / B


def _fused_body(cls_ref, key_ref, prompt_hbm, out_hbm, rs_ref,
                ids_smem, slab, in_sem, out_sem):
    L, P, LEN, C = prompt_hbm.shape
    B = out_hbm.shape[1]

    _routing(cls_ref, key_ref, ids_smem, rs_ref)

    def read_slab(l, buf):
        for kk in range(TOP_K):
            pltpu.make_async_copy(
                prompt_hbm.at[l, ids_smem[0, kk]],
                slab.at[buf, pl.ds(kk * LEN, LEN), :],
                in_sem.at[buf],
            ).start()

    def wait_slab(l, buf):
        for kk in range(TOP_K):
            pltpu.make_async_copy(
                prompt_hbm.at[l, ids_smem[0, kk]],
                slab.at[buf, pl.ds(kk * LEN, LEN), :],
                in_sem.at[buf],
            ).wait()

    def write_out(l, buf):
        for b in range(B):
            pltpu.make_async_copy(
                slab.at[buf], out_hbm.at[l, b], out_sem.at[buf],
            ).start()

    def wait_out(l, buf):
        for b in range(B):
            pltpu.make_async_copy(
                slab.at[buf], out_hbm.at[l, b], out_sem.at[buf],
            ).wait()

    read_slab(0, 0)
    for l in range(L):
        buf = l % 2
        wait_slab(l, buf)
        if l + 1 < L:
            if l > 0:
                wait_out(l - 1, 1 - buf)
            read_slab(l + 1, 1 - buf)
        write_out(l, buf)
    wait_out(L - 2, 0)
    wait_out(L - 1, 1)


def kernel(x_embed, cls_features, prompt, prompt_key):
    B = x_embed.shape[0]
    L, P, LEN, C = prompt.shape

    out, rs = pl.pallas_call(
        _fused_body,
        in_specs=[
            pl.BlockSpec(memory_space=pltpu.VMEM),
            pl.BlockSpec(memory_space=pltpu.VMEM),
            pl.BlockSpec(memory_space=pl.MemorySpace.ANY),
        ],
        out_specs=(
            pl.BlockSpec(memory_space=pl.MemorySpace.ANY),
            pl.BlockSpec(memory_space=pltpu.SMEM),
        ),
        out_shape=(
            jax.ShapeDtypeStruct((L, B, TOP_K * LEN, C), jnp.float32),
            jax.ShapeDtypeStruct((1, 1), jnp.float32),
        ),
        scratch_shapes=[
            pltpu.SMEM((1, TOP_K), jnp.int32),
            pltpu.VMEM((2, TOP_K * LEN, C), jnp.float32),
            pltpu.SemaphoreType.DMA((2,)),
            pltpu.SemaphoreType.DMA((2,)),
        ],
    )(cls_features, prompt_key, prompt)

    return out, rs.reshape(())


# speedup vs baseline: 1.1487x; 1.1487x over previous
import jax
import jax.numpy as jnp
from jax.experimental import pallas as pl
from jax.experimental.pallas import tpu as pltpu

TOP_K = 8


def _routing(cls_ref, key_ref, ids_smem, rs_ref):
    eps = 1e-12
    k = key_ref[...]
    kn = jnp.sqrt(jnp.sum(k * k, axis=1, keepdims=True))
    k_n = k / jnp.maximum(kn, eps)
    x = cls_ref[...]
    xn = jnp.sqrt(jnp.sum(x * x, axis=1, keepdims=True))
    x_n = x / jnp.maximum(xn, eps)
    sim0 = jax.lax.dot_general(
        x_n, k_n, (((1,), (1,)), ((), ())),
        preferred_element_type=jnp.float32)
    B, P = sim0.shape

    col = jax.lax.broadcasted_iota(jnp.int32, (B, P), 1)
    sim = sim0
    counts2d = jnp.zeros((B, P), jnp.int32)
    for _ in range(TOP_K):
        m = jnp.max(sim, axis=1, keepdims=True)
        cand = jnp.where(sim == m, col, P)
        j = jnp.min(cand, axis=1, keepdims=True)
        oh = col == j
        counts2d = counts2d + oh.astype(jnp.int32)
        sim = jnp.where(oh, -jnp.inf, sim)

    cnt = jnp.sum(counts2d, axis=0, keepdims=True)
    p_row = jax.lax.broadcasted_iota(jnp.int32, (1, P), 1)
    key2 = (cnt * (2 * P) + (P - 1 - p_row)).astype(jnp.float32)
    ri = jax.lax.broadcasted_iota(jnp.int32, (P, P), 0)
    ci = jax.lax.broadcasted_iota(jnp.int32, (P, P), 1)
    ident = (ri == ci).astype(jnp.float32)
    key2_col = jax.lax.dot_general(
        ident, key2, (((1,), (1,)), ((), ())),
        preferred_element_type=jnp.float32)
    gt = (key2_col > key2).astype(jnp.int32)
    rank = jnp.sum(gt, axis=0, keepdims=True)
    for j in range(TOP_K):
        ids_smem[0, j] = jnp.sum(jnp.where(rank == j, p_row, 0))
    colsum = jnp.sum(sim0, axis=0, keepdims=True)
    sel = (rank < TOP_K).astype(jnp.float32)
    rs_ref[0, 0] = jnp.sum(colsum * sel) / B


def _fused_body(cls_ref, key_ref, prompt_hbm, out_ref, rs_ref,
                ids_smem, slab, in_sem):
    L, P, LEN, C = prompt_hbm.shape
    B = out_ref.shape[1]
    l = pl.program_id(0)

    def read_slab(layer, buf):
        for kk in range(TOP_K):
            pltpu.make_async_copy(
                prompt_hbm.at[layer, ids_smem[0, kk]],
                slab.at[buf, pl.ds(kk * LEN, LEN), :],
                in_sem.at[buf],
            ).start()

    def wait_slab(buf):
        for kk in range(TOP_K):
            pltpu.make_async_copy(
                prompt_hbm.at[0, 0],
                slab.at[buf, pl.ds(kk * LEN, LEN), :],
                in_sem.at[buf],
            ).wait()

    @pl.when(l == 0)
    def _():
        _routing(cls_ref, key_ref, ids_smem, rs_ref)
        read_slab(0, 0)

    @pl.when(l + 1 < L)
    def _():
        read_slab(l + 1, (l + 1) % 2)

    wait_slab(l % 2)
    cur = slab[l % 2]
    out_ref[...] = jax.lax.broadcast_in_dim(
        cur, (1, B, TOP_K * LEN, C), (2, 3))


def kernel(x_embed, cls_features, prompt, prompt_key):
    B = x_embed.shape[0]
    L, P, LEN, C = prompt.shape

    out, rs = pl.pallas_call(
        _fused_body,
        grid=(L,),
        in_specs=[
            pl.BlockSpec((B, C), lambda l: (0, 0)),
            pl.BlockSpec((P, C), lambda l: (0, 0)),
            pl.BlockSpec(memory_space=pl.MemorySpace.ANY),
        ],
        out_specs=(
            pl.BlockSpec((1, B, TOP_K * LEN, C), lambda l: (l, 0, 0, 0)),
            pl.BlockSpec(memory_space=pltpu.SMEM),
        ),
        out_shape=(
            jax.ShapeDtypeStruct((L, B, TOP_K * LEN, C), jnp.float32),
            jax.ShapeDtypeStruct((1, 1), jnp.float32),
        ),
        scratch_shapes=[
            pltpu.SMEM((1, TOP_K), jnp.int32),
            pltpu.VMEM((2, TOP_K * LEN, C), jnp.float32),
            pltpu.SemaphoreType.DMA((2,)),
        ],
    )(cls_features, prompt_key, prompt)

    return out, rs.reshape(())
